# Initial kernel scaffold; baseline (speedup 1.0000x reference)
import jax
import jax.numpy as jnp
from jax.experimental import pallas as pl


def kernel(preds, gts, normals):
    raise NotImplementedError("write your pallas kernel here")


# speedup vs baseline: 2.4189x; 2.4189x over previous
import functools
import math

import jax
import jax.numpy as jnp
from jax import lax
from jax.experimental import pallas as pl
from jax.experimental.pallas import tpu as pltpu

KNN_K = 8
SIG_P2 = 0.03 ** 2
ANG_C = 1.0 - math.cos(math.radians(15))
MB = 256


def _topk_body(pred_ref, gts_ref, dist_ref, idx_ref):
    p = pred_ref[0]
    g = gts_ref[0]
    dx = p[:, 0:1] - g[0:1, :]
    dy = p[:, 1:2] - g[1:2, :]
    dz = p[:, 2:3] - g[2:3, :]
    d = dx * dx + dy * dy + dz * dz
    n = d.shape[1]
    iota = lax.broadcasted_iota(jnp.int32, d.shape, 1)
    big = jnp.float32(jnp.inf)
    for j in range(KNN_K):
        m = jnp.min(d, axis=1, keepdims=True)
        idx2 = jnp.where(d == m, iota, n)
        am = jnp.min(idx2, axis=1, keepdims=True)
        dist_ref[0, :, j:j + 1] = m
        idx_ref[0, :, j:j + 1] = am
        d = jnp.where(idx2 == am, big, d)


def _topk(preds, gts_t, interpret=False):
    B, M, _ = preds.shape
    N = gts_t.shape[2]
    grid = (B, M // MB)
    return pl.pallas_call(
        _topk_body,
        grid=grid,
        in_specs=[
            pl.BlockSpec((1, MB, 3), lambda b, i: (b, i, 0)),
            pl.BlockSpec((1, 3, N), lambda b, i: (b, 0, 0)),
        ],
        out_specs=[
            pl.BlockSpec((1, MB, KNN_K), lambda b, i: (b, i, 0)),
            pl.BlockSpec((1, MB, KNN_K), lambda b, i: (b, i, 0)),
        ],
        out_shape=[
            jax.ShapeDtypeStruct((B, M, KNN_K), jnp.float32),
            jax.ShapeDtypeStruct((B, M, KNN_K), jnp.int32),
        ],
        interpret=interpret,
    )(preds, gts_t)


def _stage2_jax(preds, gts, normals, knn_dist, knn_idx):
    B = preds.shape[0]
    b = jnp.arange(B)[:, None, None]
    nb_points = gts[b, knn_idx]
    nb_normals = normals[b, knn_idx]
    distance_w = jnp.exp(-knn_dist / SIG_P2)
    estm_normal = nb_normals[:, :, 0:1, :]
    inner = jnp.sum(nb_normals * estm_normal, axis=-1)
    angle_w = jnp.exp(-(1.0 - inner) / ANG_C)
    weights = distance_w * angle_w
    inner_prod = jnp.sum((preds[:, :, None, :] - nb_points) * nb_normals, axis=-1)
    inner_prod = jnp.abs(inner_prod)
    disp = jnp.sum(inner_prod * weights, axis=-1) / jnp.sum(weights, axis=-1)
    return jnp.sum(disp)


def kernel(preds, gts, normals):
    gts_t = jnp.transpose(gts, (0, 2, 1))
    knn_dist, knn_idx = _topk(preds, gts_t)
    return _stage2_jax(preds, gts, normals, knn_dist, knn_idx)


# speedup vs baseline: 30.6880x; 12.6865x over previous
import functools
import math

import jax
import jax.numpy as jnp
from jax import lax
from jax.experimental import pallas as pl
from jax.experimental.pallas import tpu as pltpu
from jax.experimental.pallas import tpu_sc as plsc

KNN_K = 8
SIG_P2 = 0.03 ** 2
ANG_C = 1.0 - math.cos(math.radians(15))
MB = 256


def _topk_body(pred_ref, gts_ref, dist_ref, idx_ref):
    p = pred_ref[0]
    g = gts_ref[0]
    dx = p[:, 0:1] - g[0:1, :]
    dy = p[:, 1:2] - g[1:2, :]
    dz = p[:, 2:3] - g[2:3, :]
    d = dx * dx + dy * dy + dz * dz
    n = d.shape[1]
    iota = lax.broadcasted_iota(jnp.int32, d.shape, 1)
    big = jnp.float32(jnp.inf)
    for j in range(KNN_K):
        m = jnp.min(d, axis=1, keepdims=True)
        idx2 = jnp.where(d == m, iota, n)
        am = jnp.min(idx2, axis=1, keepdims=True)
        dist_ref[0, :, j:j + 1] = m
        idx_ref[0, :, j:j + 1] = am
        d = jnp.where(idx2 == am, big, d)


def _topk(preds, gts_t, interpret=False):
    B, M, _ = preds.shape
    N = gts_t.shape[2]
    grid = (B, M // MB)
    return pl.pallas_call(
        _topk_body,
        grid=grid,
        in_specs=[
            pl.BlockSpec((1, MB, 3), lambda b, i: (b, i, 0)),
            pl.BlockSpec((1, 3, N), lambda b, i: (b, 0, 0)),
        ],
        out_specs=[
            pl.BlockSpec((1, MB, KNN_K), lambda b, i: (b, i, 0)),
            pl.BlockSpec((1, MB, KNN_K), lambda b, i: (b, i, 0)),
        ],
        out_shape=[
            jax.ShapeDtypeStruct((B, M, KNN_K), jnp.float32),
            jax.ShapeDtypeStruct((B, M, KNN_K), jnp.int32),
        ],
        interpret=interpret,
    )(preds, gts_t)


def _make_sc_stage2(B, M, N, interpret=False):
    NC, NS = 2, 16
    NW = NC * NS
    QPW = (B * M) // NW
    WPB = M // QPW
    NG = QPW // 16
    mesh = plsc.VectorSubcoreMesh(core_axis_name="c", subcore_axis_name="s")

    @functools.partial(
        pl.kernel,
        mesh=mesh,
        out_type=jax.ShapeDtypeStruct((NW * 16,), jnp.float32),
        scratch_types=[
            pltpu.VMEM((N,), jnp.float32),
            pltpu.VMEM((N,), jnp.float32),
            pltpu.VMEM((N,), jnp.float32),
            pltpu.VMEM((N,), jnp.float32),
            pltpu.VMEM((N,), jnp.float32),
            pltpu.VMEM((N,), jnp.float32),
            pltpu.VMEM((QPW,), jnp.float32),
            pltpu.VMEM((QPW,), jnp.float32),
            pltpu.VMEM((QPW,), jnp.float32),
            pltpu.VMEM((KNN_K, QPW), jnp.float32),
            pltpu.VMEM((KNN_K, QPW), jnp.int32),
            pltpu.VMEM((16,), jnp.float32),
        ],
        compiler_params=pltpu.CompilerParams(needs_layout_passes=False),
        interpret=interpret,
    )
    def sc2(dist_hbm, idx_hbm, preds_hbm, gts_hbm, nrm_hbm, out_hbm,
            gx, gy, gz, nx, ny, nz, pxv, pyv, pzv, dv, iv, av):
        wid = lax.axis_index("s") * NC + lax.axis_index("c")
        b = wid // WPB
        qoff = (wid % WPB) * QPW
        pltpu.sync_copy(gts_hbm.at[pl.ds((b * 3 + 0) * N, N)], gx)
        pltpu.sync_copy(gts_hbm.at[pl.ds((b * 3 + 1) * N, N)], gy)
        pltpu.sync_copy(gts_hbm.at[pl.ds((b * 3 + 2) * N, N)], gz)
        pltpu.sync_copy(nrm_hbm.at[pl.ds((b * 3 + 0) * N, N)], nx)
        pltpu.sync_copy(nrm_hbm.at[pl.ds((b * 3 + 1) * N, N)], ny)
        pltpu.sync_copy(nrm_hbm.at[pl.ds((b * 3 + 2) * N, N)], nz)
        pltpu.sync_copy(preds_hbm.at[pl.ds((b * 3 + 0) * M + qoff, QPW)], pxv)
        pltpu.sync_copy(preds_hbm.at[pl.ds((b * 3 + 1) * M + qoff, QPW)], pyv)
        pltpu.sync_copy(preds_hbm.at[pl.ds((b * 3 + 2) * M + qoff, QPW)], pzv)
        pltpu.sync_copy(
            dist_hbm.at[pl.ds(b * KNN_K, KNN_K), pl.ds(qoff, QPW)], dv)
        pltpu.sync_copy(
            idx_hbm.at[pl.ds(b * KNN_K, KNN_K), pl.ds(qoff, QPW)], iv)

        def body(g, acc):
            goff = g * 16
            px = pxv[pl.ds(goff, 16)]
            py = pyv[pl.ds(goff, 16)]
            pz = pzv[pl.ds(goff, 16)]
            n0x = n0y = n0z = None
            num = jnp.zeros((16,), jnp.float32)
            den = jnp.zeros((16,), jnp.float32)
            for j in range(KNN_K):
                dj = dv[j, pl.ds(goff, 16)]
                ij = iv[j, pl.ds(goff, 16)]
                gxj = plsc.load_gather(gx, [ij])
                gyj = plsc.load_gather(gy, [ij])
                gzj = plsc.load_gather(gz, [ij])
                nxj = plsc.load_gather(nx, [ij])
                nyj = plsc.load_gather(ny, [ij])
                nzj = plsc.load_gather(nz, [ij])
                if j == 0:
                    n0x, n0y, n0z = nxj, nyj, nzj
                cos = nxj * n0x + nyj * n0y + nzj * n0z
                aw = jnp.exp(-(1.0 - cos) / ANG_C)
                dw = jnp.exp(-dj / SIG_P2)
                w = dw * aw
                ip = jnp.abs((px - gxj) * nxj + (py - gyj) * nyj
                             + (pz - gzj) * nzj)
                num = num + ip * w
                den = den + w
            return acc + num / den

        acc = lax.fori_loop(0, NG, body, jnp.zeros((16,), jnp.float32))
        av[...] = acc
        pltpu.sync_copy(av, out_hbm.at[pl.ds(wid * 16, 16)])

    return sc2


def kernel(preds, gts, normals):
    B, M, _ = preds.shape
    N = gts.shape[1]
    gts_t = jnp.transpose(gts, (0, 2, 1))
    nrm_flat = jnp.transpose(normals, (0, 2, 1)).reshape(-1)
    preds_flat = jnp.transpose(preds, (0, 2, 1)).reshape(-1)
    knn_dist, knn_idx = _topk(preds, gts_t)
    dist2 = jnp.transpose(knn_dist, (0, 2, 1)).reshape(B * KNN_K, M)
    idx2 = jnp.transpose(knn_idx, (0, 2, 1)).reshape(B * KNN_K, M)
    sc2 = _make_sc_stage2(B, M, N)
    partials = sc2(dist2, idx2, preds_flat, gts_t.reshape(-1), nrm_flat)
    return jnp.sum(partials)


# speedup vs baseline: 35.6277x; 1.1610x over previous
import functools
import math

import jax
import jax.numpy as jnp
from jax import lax
from jax.experimental import pallas as pl
from jax.experimental.pallas import tpu as pltpu
from jax.experimental.pallas import tpu_sc as plsc

KNN_K = 8
SIG_P2 = 0.03 ** 2
ANG_C = 1.0 - math.cos(math.radians(15))
MB = 256


def _topk_body(pred_ref, gts_ref, dist_ref, idx_ref):
    p = pred_ref[0]
    g = gts_ref[0]
    dx = p[:, 0:1] - g[0:1, :]
    dy = p[:, 1:2] - g[1:2, :]
    dz = p[:, 2:3] - g[2:3, :]
    d = dx * dx + dy * dy + dz * dz
    n = d.shape[1]
    iota_f = lax.broadcasted_iota(jnp.int32, d.shape, 1).astype(jnp.float32)
    big = jnp.float32(jnp.inf)
    bign = jnp.float32(n)
    for j in range(KNN_K):
        m = jnp.min(d, axis=1, keepdims=True)
        eq = d == m
        idxc = jnp.where(eq, iota_f, bign)
        am = jnp.min(idxc, axis=1, keepdims=True)
        dist_ref[0, :, j:j + 1] = m
        idx_ref[0, :, j:j + 1] = am.astype(jnp.int32)
        d = jnp.where(eq, big, d)


def _topk(preds, gts_t, interpret=False):
    B, M, _ = preds.shape
    N = gts_t.shape[2]
    grid = (B, M // MB)
    return pl.pallas_call(
        _topk_body,
        grid=grid,
        in_specs=[
            pl.BlockSpec((1, MB, 3), lambda b, i: (b, i, 0)),
            pl.BlockSpec((1, 3, N), lambda b, i: (b, 0, 0)),
        ],
        out_specs=[
            pl.BlockSpec((1, MB, KNN_K), lambda b, i: (b, i, 0)),
            pl.BlockSpec((1, MB, KNN_K), lambda b, i: (b, i, 0)),
        ],
        out_shape=[
            jax.ShapeDtypeStruct((B, M, KNN_K), jnp.float32),
            jax.ShapeDtypeStruct((B, M, KNN_K), jnp.int32),
        ],
        interpret=interpret,
    )(preds, gts_t)


def _make_sc_stage2(B, M, N):
    NC, NS = 2, 16
    NW = NC * NS
    QPW = (B * M) // NW
    WPB = M // QPW
    NG = QPW // 16
    mesh = plsc.VectorSubcoreMesh(core_axis_name="c", subcore_axis_name="s")

    @functools.partial(
        pl.kernel,
        mesh=mesh,
        out_type=jax.ShapeDtypeStruct((NW * 16,), jnp.float32),
        scratch_types=[
            pltpu.VMEM((3 * N,), jnp.float32),
            pltpu.VMEM((3 * N,), jnp.float32),
            pltpu.VMEM((3 * QPW,), jnp.float32),
            pltpu.VMEM((KNN_K * QPW,), jnp.float32),
            pltpu.VMEM((KNN_K * QPW,), jnp.int32),
            pltpu.VMEM((16,), jnp.float32),
        ],
        compiler_params=pltpu.CompilerParams(needs_layout_passes=False),
    )
    def sc2(dist_hbm, idx_hbm, preds_hbm, gts_hbm, nrm_hbm, out_hbm,
            gt_v, nr_v, pf_v, df_v, if_v, av):
        wid = lax.axis_index("s") * NC + lax.axis_index("c")
        b = wid // WPB
        qoff = (wid % WPB) * QPW
        pltpu.sync_copy(gts_hbm.at[pl.ds(b * 3 * N, 3 * N)], gt_v)
        pltpu.sync_copy(nrm_hbm.at[pl.ds(b * 3 * N, 3 * N)], nr_v)
        pltpu.sync_copy(preds_hbm.at[pl.ds((b * M + qoff) * 3, 3 * QPW)], pf_v)
        pltpu.sync_copy(
            dist_hbm.at[pl.ds((b * M + qoff) * KNN_K, KNN_K * QPW)], df_v)
        pltpu.sync_copy(
            idx_hbm.at[pl.ds((b * M + qoff) * KNN_K, KNN_K * QPW)], if_v)
        lanes = jnp.arange(16, dtype=jnp.int32)

        def body(g, acc):
            ivp = g * 48 + lanes * 3
            px = plsc.load_gather(pf_v, [ivp])
            py = plsc.load_gather(pf_v, [ivp + 1])
            pz = plsc.load_gather(pf_v, [ivp + 2])
            ivd = g * (16 * KNN_K) + lanes * KNN_K
            n0x = n0y = n0z = None
            num = jnp.zeros((16,), jnp.float32)
            den = jnp.zeros((16,), jnp.float32)
            for j in range(KNN_K):
                dj = plsc.load_gather(df_v, [ivd + j])
                ij = plsc.load_gather(if_v, [ivd + j])
                i3 = ij * 3
                gxj = plsc.load_gather(gt_v, [i3])
                gyj = plsc.load_gather(gt_v, [i3 + 1])
                gzj = plsc.load_gather(gt_v, [i3 + 2])
                nxj = plsc.load_gather(nr_v, [i3])
                nyj = plsc.load_gather(nr_v, [i3 + 1])
                nzj = plsc.load_gather(nr_v, [i3 + 2])
                if j == 0:
                    n0x, n0y, n0z = nxj, nyj, nzj
                cos = nxj * n0x + nyj * n0y + nzj * n0z
                aw = jnp.exp(-(1.0 - cos) / ANG_C)
                dw = jnp.exp(-dj / SIG_P2)
                w = dw * aw
                ip = jnp.abs((px - gxj) * nxj + (py - gyj) * nyj
                             + (pz - gzj) * nzj)
                num = num + ip * w
                den = den + w
            return acc + num / den

        acc = lax.fori_loop(0, NG, body, jnp.zeros((16,), jnp.float32))
        av[...] = acc
        pltpu.sync_copy(av, out_hbm.at[pl.ds(wid * 16, 16)])

    return sc2


def kernel(preds, gts, normals):
    B, M, _ = preds.shape
    N = gts.shape[1]
    gts_t = jnp.transpose(gts, (0, 2, 1))
    knn_dist, knn_idx = _topk(preds, gts_t)
    sc2 = _make_sc_stage2(B, M, N)
    partials = sc2(knn_dist.reshape(-1), knn_idx.reshape(-1),
                   preds.reshape(-1), gts.reshape(-1), normals.reshape(-1))
    return jnp.sum(partials)


# speedup vs baseline: 36.4445x; 1.0229x over previous
import functools
import math

import jax
import jax.numpy as jnp
from jax import lax
from jax.experimental import pallas as pl
from jax.experimental.pallas import tpu as pltpu
from jax.experimental.pallas import tpu_sc as plsc

KNN_K = 8
SIG_P2 = 0.03 ** 2
ANG_C = 1.0 - math.cos(math.radians(15))
MB = 512


def _topk_body(pred_ref, gts_ref, dist_ref, idx_ref):
    p = pred_ref[0]
    g = gts_ref[0]
    dx = p[:, 0:1] - g[0:1, :]
    dy = p[:, 1:2] - g[1:2, :]
    dz = p[:, 2:3] - g[2:3, :]
    d = dx * dx + dy * dy + dz * dz
    n = d.shape[1]
    iota_f = lax.broadcasted_iota(jnp.int32, d.shape, 1).astype(jnp.float32)
    big = jnp.float32(jnp.inf)
    bign = jnp.float32(n)
    for j in range(KNN_K):
        m = jnp.min(d, axis=1, keepdims=True)
        eq = d == m
        idxc = jnp.where(eq, iota_f, bign)
        am = jnp.min(idxc, axis=1, keepdims=True)
        dist_ref[0, :, j:j + 1] = m
        idx_ref[0, :, j:j + 1] = am.astype(jnp.int32)
        d = jnp.where(eq, big, d)


def _topk(preds, gts_t, interpret=False):
    B, M, _ = preds.shape
    N = gts_t.shape[2]
    grid = (B, M // MB)
    return pl.pallas_call(
        _topk_body,
        grid=grid,
        in_specs=[
            pl.BlockSpec((1, MB, 3), lambda b, i: (b, i, 0)),
            pl.BlockSpec((1, 3, N), lambda b, i: (b, 0, 0)),
        ],
        out_specs=[
            pl.BlockSpec((1, MB, KNN_K), lambda b, i: (b, i, 0)),
            pl.BlockSpec((1, MB, KNN_K), lambda b, i: (b, i, 0)),
        ],
        out_shape=[
            jax.ShapeDtypeStruct((B, M, KNN_K), jnp.float32),
            jax.ShapeDtypeStruct((B, M, KNN_K), jnp.int32),
        ],
        interpret=interpret,
    )(preds, gts_t)


def _make_sc_stage2(B, M, N):
    NC, NS = 2, 16
    NW = NC * NS
    QPW = (B * M) // NW
    WPB = M // QPW
    NG = QPW // 16
    mesh = plsc.VectorSubcoreMesh(core_axis_name="c", subcore_axis_name="s")

    @functools.partial(
        pl.kernel,
        mesh=mesh,
        out_type=jax.ShapeDtypeStruct((NW * 16,), jnp.float32),
        scratch_types=[
            pltpu.VMEM((3 * N,), jnp.float32),
            pltpu.VMEM((3 * N,), jnp.float32),
            pltpu.VMEM((3 * QPW,), jnp.float32),
            pltpu.VMEM((KNN_K * QPW,), jnp.float32),
            pltpu.VMEM((KNN_K * QPW,), jnp.int32),
            pltpu.VMEM((16,), jnp.float32),
        ],
        compiler_params=pltpu.CompilerParams(needs_layout_passes=False),
    )
    def sc2(dist_hbm, idx_hbm, preds_hbm, gts_hbm, nrm_hbm, out_hbm,
            gt_v, nr_v, pf_v, df_v, if_v, av):
        wid = lax.axis_index("s") * NC + lax.axis_index("c")
        b = wid // WPB
        qoff = (wid % WPB) * QPW
        pltpu.sync_copy(gts_hbm.at[pl.ds(b * 3 * N, 3 * N)], gt_v)
        pltpu.sync_copy(nrm_hbm.at[pl.ds(b * 3 * N, 3 * N)], nr_v)
        pltpu.sync_copy(preds_hbm.at[pl.ds((b * M + qoff) * 3, 3 * QPW)], pf_v)
        pltpu.sync_copy(
            dist_hbm.at[pl.ds((b * M + qoff) * KNN_K, KNN_K * QPW)], df_v)
        pltpu.sync_copy(
            idx_hbm.at[pl.ds((b * M + qoff) * KNN_K, KNN_K * QPW)], if_v)
        lanes = jnp.arange(16, dtype=jnp.int32)

        def body(g, acc):
            ivp = g * 48 + lanes * 3
            px = plsc.load_gather(pf_v, [ivp])
            py = plsc.load_gather(pf_v, [ivp + 1])
            pz = plsc.load_gather(pf_v, [ivp + 2])
            ivd = g * (16 * KNN_K) + lanes * KNN_K
            n0x = n0y = n0z = None
            num = jnp.zeros((16,), jnp.float32)
            den = jnp.zeros((16,), jnp.float32)
            for j in range(KNN_K):
                dj = plsc.load_gather(df_v, [ivd + j])
                ij = plsc.load_gather(if_v, [ivd + j])
                i3 = ij * 3
                gxj = plsc.load_gather(gt_v, [i3])
                gyj = plsc.load_gather(gt_v, [i3 + 1])
                gzj = plsc.load_gather(gt_v, [i3 + 2])
                nxj = plsc.load_gather(nr_v, [i3])
                nyj = plsc.load_gather(nr_v, [i3 + 1])
                nzj = plsc.load_gather(nr_v, [i3 + 2])
                if j == 0:
                    n0x, n0y, n0z = nxj, nyj, nzj
                cos = nxj * n0x + nyj * n0y + nzj * n0z
                aw = jnp.exp(-(1.0 - cos) / ANG_C)
                dw = jnp.exp(-dj / SIG_P2)
                w = dw * aw
                ip = jnp.abs((px - gxj) * nxj + (py - gyj) * nyj
                             + (pz - gzj) * nzj)
                num = num + ip * w
                den = den + w
            return acc + num / den

        acc = lax.fori_loop(0, NG, body, jnp.zeros((16,), jnp.float32))
        av[...] = acc
        pltpu.sync_copy(av, out_hbm.at[pl.ds(wid * 16, 16)])

    return sc2


def kernel(preds, gts, normals):
    B, M, _ = preds.shape
    N = gts.shape[1]
    gts_t = jnp.transpose(gts, (0, 2, 1))
    knn_dist, knn_idx = _topk(preds, gts_t)
    sc2 = _make_sc_stage2(B, M, N)
    partials = sc2(knn_dist.reshape(-1), knn_idx.reshape(-1),
                   preds.reshape(-1), gts.reshape(-1), normals.reshape(-1))
    return jnp.sum(partials)
